# baseline (speedup 1.0000x reference)
import functools

import jax
import jax.numpy as jnp
from jax.experimental import pallas as pl
from jax.experimental.pallas import tpu as pltpu

_MiB = 1024 * 1024


def _matmul_dup_kernel(x_ref, wt_ref, b_ref, o_ref, *, d):
    xb = x_ref[...].astype(jnp.bfloat16)
    y = jnp.dot(xb, wt_ref[...], preferred_element_type=jnp.float32)
    y = y + b_ref[...]
    o_ref[:, :d] = y
    o_ref[:, d:] = y


def kernel(x, weight, bias):
    B, S, D = x.shape
    rows = B * S
    scale = 2

    tm = 8
    for cand in (512, 256, 128, 64, 32, 16, 8):
        if rows % cand == 0:
            tm = cand
            break

    x2d = x.reshape(rows, D)
    wt = weight.T.astype(jnp.bfloat16)
    b2d = bias.astype(jnp.float32).reshape(1, D)

    body = functools.partial(_matmul_dup_kernel, d=D)
    out2d = pl.pallas_call(
        body,
        out_shape=jax.ShapeDtypeStruct((rows, scale * D), x.dtype),
        grid=(rows // tm,),
        in_specs=[
            pl.BlockSpec((tm, D), lambda i: (i, 0)),
            pl.BlockSpec((D, D), lambda i: (0, 0)),
            pl.BlockSpec((1, D), lambda i: (0, 0)),
        ],
        out_specs=pl.BlockSpec((tm, scale * D), lambda i: (i, 0)),
        compiler_params=pltpu.CompilerParams(
            dimension_semantics=("parallel",),
            vmem_limit_bytes=96 * _MiB,
        ),
    )(x2d, wt, b2d)

    return out2d.reshape(rows, scale, D).reshape(B, S * scale, D)


# speedup vs baseline: 1.0771x; 1.0771x over previous
import functools

import jax
import jax.numpy as jnp
from jax.experimental import pallas as pl
from jax.experimental.pallas import tpu as pltpu

_MiB = 1024 * 1024


def _pipelined_body(x_hbm, wt_ref, b_ref, o_hbm,
                    x_vmem, y_ref, rd_sems, out_sems,
                    *, n_tiles, tm, d, n_slots):
    def rd_copy(i):
        sl = pl.ds(i * tm, tm)
        return pltpu.make_async_copy(x_hbm.at[sl, :], x_vmem.at[sl, :],
                                     rd_sems.at[i])

    def out_copy(i):
        slot = i % n_slots
        return pltpu.make_async_copy(y_ref.at[slot],
                                     o_hbm.at[pl.ds(i * tm, tm), :],
                                     out_sems.at[slot])

    for i in range(n_tiles):
        rd_copy(i).start()

    for i in range(n_tiles):
        rd_copy(i).wait()
        if i >= n_slots:
            out_copy(i - n_slots).wait()
        slot = i % n_slots
        xb = x_vmem[pl.ds(i * tm, tm), :].astype(jnp.bfloat16)
        y = jnp.dot(xb, wt_ref[...], preferred_element_type=jnp.float32)
        y = y + b_ref[...]
        y_ref[slot, :, :d] = y
        y_ref[slot, :, d:] = y
        out_copy(i).start()

    for i in range(max(0, n_tiles - n_slots), n_tiles):
        out_copy(i).wait()


def kernel(x, weight, bias):
    B, S, D = x.shape
    rows = B * S
    scale = 2

    tm = 8
    for cand in (512, 256, 128, 64, 32, 16, 8):
        if rows % cand == 0:
            tm = cand
            break
    n_tiles = rows // tm
    n_slots = min(3, n_tiles)

    x2d = x.reshape(rows, D)
    wt = weight.T.astype(jnp.bfloat16)
    b2d = bias.astype(jnp.float32).reshape(1, D)

    body = functools.partial(_pipelined_body, n_tiles=n_tiles, tm=tm, d=D,
                             n_slots=n_slots)
    out2d = pl.pallas_call(
        body,
        out_shape=jax.ShapeDtypeStruct((rows, scale * D), x.dtype),
        in_specs=[
            pl.BlockSpec(memory_space=pl.ANY),
            pl.BlockSpec(memory_space=pltpu.VMEM),
            pl.BlockSpec(memory_space=pltpu.VMEM),
        ],
        out_specs=pl.BlockSpec(memory_space=pl.ANY),
        scratch_shapes=[
            pltpu.VMEM((rows, D), jnp.float32),
            pltpu.VMEM((n_slots, tm, scale * D), jnp.float32),
            pltpu.SemaphoreType.DMA((n_tiles,)),
            pltpu.SemaphoreType.DMA((n_slots,)),
        ],
        compiler_params=pltpu.CompilerParams(
            vmem_limit_bytes=56 * _MiB,
        ),
    )(x2d, wt, b2d)

    return out2d.reshape(rows, scale, D).reshape(B, S * scale, D)


# speedup vs baseline: 1.0968x; 1.0182x over previous
import functools

import jax
import jax.numpy as jnp
from jax.experimental import pallas as pl
from jax.experimental.pallas import tpu as pltpu

_MiB = 1024 * 1024


def _pipelined_body(x_hbm, wt_ref, b_ref, o_hbm,
                    x_vmem, y_ref, rd_sems, out_sems,
                    *, n_tiles, tm, d, n_slots):
    def rd_copy(i):
        sl = pl.ds(i * tm, tm)
        return pltpu.make_async_copy(x_hbm.at[sl, :], x_vmem.at[sl, :],
                                     rd_sems.at[i])

    def out_copy(i):
        slot = i % n_slots
        return pltpu.make_async_copy(y_ref.at[slot],
                                     o_hbm.at[pl.ds(i * tm, tm), :],
                                     out_sems.at[slot])

    for i in range(n_tiles):
        rd_copy(i).start()

    for i in range(n_tiles):
        rd_copy(i).wait()
        if i >= n_slots:
            out_copy(i - n_slots).wait()
        slot = i % n_slots
        xt = x_vmem[pl.ds(i * tm, tm), :]
        y = jax.lax.dot_general(xt, wt_ref[...],
                                dimension_numbers=(((1,), (1,)), ((), ())),
                                preferred_element_type=jnp.float32)
        y = y + b_ref[...]
        y_ref[slot, :, :d] = y
        y_ref[slot, :, d:] = y
        out_copy(i).start()

    for i in range(max(0, n_tiles - n_slots), n_tiles):
        out_copy(i).wait()


def kernel(x, weight, bias):
    B, S, D = x.shape
    rows = B * S
    scale = 2

    tm = 8
    for cand in (512, 256, 128, 64, 32, 16, 8):
        if rows % cand == 0:
            tm = cand
            break
    n_tiles = rows // tm
    n_slots = min(3, n_tiles)

    x2d = x.reshape(rows, D)
    wt = weight
    b2d = bias.astype(jnp.float32).reshape(1, D)

    body = functools.partial(_pipelined_body, n_tiles=n_tiles, tm=tm, d=D,
                             n_slots=n_slots)
    out2d = pl.pallas_call(
        body,
        out_shape=jax.ShapeDtypeStruct((rows, scale * D), x.dtype),
        in_specs=[
            pl.BlockSpec(memory_space=pl.ANY),
            pl.BlockSpec(memory_space=pltpu.VMEM),
            pl.BlockSpec(memory_space=pltpu.VMEM),
        ],
        out_specs=pl.BlockSpec(memory_space=pl.ANY),
        scratch_shapes=[
            pltpu.VMEM((rows, D), jnp.float32),
            pltpu.VMEM((n_slots, tm, scale * D), jnp.float32),
            pltpu.SemaphoreType.DMA((n_tiles,)),
            pltpu.SemaphoreType.DMA((n_slots,)),
        ],
        compiler_params=pltpu.CompilerParams(
            vmem_limit_bytes=56 * _MiB,
        ),
    )(x2d, wt, b2d)

    return out2d.reshape(rows, scale, D).reshape(B, S * scale, D)
